# baseline (device time: 85838 ns/iter reference)
import jax
import jax.numpy as jnp
from jax import lax
from jax.experimental import pallas as pl
from jax.experimental.pallas import tpu as pltpu

N_DEV = 16
B = 2
SQ = 128
HQ = 8
DH = 64
D = 512


def kernel(x, Wq, Wo, K_ext, V_ext):
    def body(x_ref, wq_ref, wo_ref, k_ref, v_ref, out_ref,
             parts_o_ref, parts_s_ref, o_send, o_recv, s_send, s_recv):
        my = lax.axis_index("i")
        left = (my - 1) % N_DEV
        right = (my + 1) % N_DEV

        barrier = pltpu.get_barrier_semaphore()
        for nbr in (left, right):
            pl.semaphore_signal(barrier, inc=1, device_id=(nbr,),
                                device_id_type=pl.DeviceIdType.MESH)
        pl.semaphore_wait(barrier, 2)

        x2 = jnp.reshape(x_ref[:], (B * SQ, D)).astype(jnp.bfloat16)
        wq = wq_ref[:].astype(jnp.bfloat16)
        q2 = lax.dot_general(x2, wq, (((1,), (0,)), ((), ())),
                             preferred_element_type=jnp.float32)
        q2 = (q2 * 0.125).astype(jnp.bfloat16)

        for b in range(B):
            for h in range(HQ):
                qbh = q2[b * SQ:(b + 1) * SQ, h * DH:(h + 1) * DH]
                kbh = k_ref[b, :, h, :].astype(jnp.bfloat16)
                vbh = v_ref[b, :, h, :].astype(jnp.bfloat16)
                s2 = lax.dot_general(kbh, qbh, (((1,), (1,)), ((), ())),
                                     preferred_element_type=jnp.float32)
                m = jnp.max(s2, axis=0, keepdims=True)
                p2 = jnp.exp(s2 - m)
                l = jnp.sum(p2, axis=0, keepdims=True)
                o_t = lax.dot_general(vbh, p2.astype(jnp.bfloat16),
                                      (((0,), (0,)), ((), ())),
                                      preferred_element_type=jnp.float32)
                parts_o_ref[0, b, h] = o_t.astype(jnp.bfloat16)
                parts_s_ref[0, b, h, 0:1, :] = m
                parts_s_ref[0, b, h, 1:2, :] = l

        for h in range(N_DEV - 1):
            r_o = pltpu.make_async_remote_copy(
                src_ref=parts_o_ref.at[h], dst_ref=parts_o_ref.at[h + 1],
                send_sem=o_send.at[h], recv_sem=o_recv.at[h],
                device_id=(right,), device_id_type=pl.DeviceIdType.MESH)
            r_s = pltpu.make_async_remote_copy(
                src_ref=parts_s_ref.at[h], dst_ref=parts_s_ref.at[h + 1],
                send_sem=s_send.at[h], recv_sem=s_recv.at[h],
                device_id=(right,), device_id_type=pl.DeviceIdType.MESH)
            r_o.start()
            r_s.start()
            r_o.wait()
            r_s.wait()

        acc_o = parts_o_ref[0].astype(jnp.float32)
        acc_m = parts_s_ref[0, :, :, 0:1, :]
        acc_l = parts_s_ref[0, :, :, 1:2, :]
        for k in range(1, N_DEV):
            o_k = parts_o_ref[k].astype(jnp.float32)
            m_k = parts_s_ref[k, :, :, 0:1, :]
            l_k = parts_s_ref[k, :, :, 1:2, :]
            m_new = jnp.maximum(acc_m, m_k)
            a_w = jnp.exp(acc_m - m_new)
            b_w = jnp.exp(m_k - m_new)
            acc_l = acc_l * a_w + l_k * b_w
            acc_o = acc_o * a_w + o_k * b_w
            acc_m = m_new

        attn_t = jnp.reshape((acc_o / acc_l).astype(jnp.bfloat16),
                             (B, HQ * DH, SQ))
        wo = wo_ref[:].astype(jnp.bfloat16)
        for b in range(B):
            out_ref[b] = lax.dot_general(attn_t[b], wo,
                                         (((0,), (0,)), ((), ())),
                                         preferred_element_type=jnp.float32)

    return pl.pallas_call(
        body,
        out_shape=jax.ShapeDtypeStruct((B, SQ, D), jnp.float32),
        in_specs=[pl.BlockSpec(memory_space=pltpu.VMEM)] * 5,
        out_specs=pl.BlockSpec(memory_space=pltpu.VMEM),
        scratch_shapes=[
            pltpu.VMEM((N_DEV, B, HQ, DH, SQ), jnp.bfloat16),
            pltpu.VMEM((N_DEV, B, HQ, 2, SQ), jnp.float32),
            pltpu.SemaphoreType.DMA((N_DEV - 1,)),
            pltpu.SemaphoreType.DMA((N_DEV - 1,)),
            pltpu.SemaphoreType.DMA((N_DEV - 1,)),
            pltpu.SemaphoreType.DMA((N_DEV - 1,)),
        ],
        compiler_params=pltpu.CompilerParams(collective_id=0),
    )(x, Wq, Wo, K_ext, V_ext)


# device time: 37360 ns/iter; 2.2976x vs baseline; 2.2976x over previous
import functools

import jax
import jax.numpy as jnp
from jax import lax
from jax.experimental import pallas as pl
from jax.experimental.pallas import tpu as pltpu

N_DEV = 16
N_STEPS = 4
B = 2
SQ = 128
HQ = 8
DH = 64
D = 512


def kernel(x, Wq, Wo, K_ext, V_ext):
    def body(x_ref, wq_ref, wo_ref, k_ref, v_ref, out_ref,
             acc_o_ref, acc_s_ref, send_o_ref, recv_o_ref,
             send_s_ref, recv_s_ref, o_send, o_recv, s_send, s_recv):
        my = lax.axis_index("i")
        partners = [my ^ (1 << k) for k in range(N_STEPS)]

        barrier = pltpu.get_barrier_semaphore()
        for p in partners:
            pl.semaphore_signal(barrier, inc=1, device_id=(p,),
                                device_id_type=pl.DeviceIdType.MESH)
        pl.semaphore_wait(barrier, N_STEPS)

        x2 = jnp.reshape(x_ref[:], (B * SQ, D)).astype(jnp.bfloat16)
        wq = wq_ref[:].astype(jnp.bfloat16)
        q2 = lax.dot_general(x2, wq, (((1,), (0,)), ((), ())),
                             preferred_element_type=jnp.float32)
        q2 = (q2 * 0.125).astype(jnp.bfloat16)

        for b in range(B):
            for h in range(HQ):
                qbh = q2[b * SQ:(b + 1) * SQ, h * DH:(h + 1) * DH]
                kbh = k_ref[b, :, h, :].astype(jnp.bfloat16)
                vbh = v_ref[b, :, h, :].astype(jnp.bfloat16)
                s2 = lax.dot_general(kbh, qbh, (((1,), (1,)), ((), ())),
                                     preferred_element_type=jnp.float32)
                m = jnp.max(s2, axis=0, keepdims=True)
                p2 = jnp.exp(s2 - m)
                l = jnp.sum(p2, axis=0, keepdims=True)
                o_t = lax.dot_general(vbh, p2.astype(jnp.bfloat16),
                                      (((0,), (0,)), ((), ())),
                                      preferred_element_type=jnp.float32)
                acc_o_ref[b, h] = o_t
                acc_s_ref[b, h, 0:1, :] = m
                acc_s_ref[b, h, 1:2, :] = l

        for k in range(N_STEPS):
            send_o_ref[k] = acc_o_ref[:].astype(jnp.bfloat16)
            send_s_ref[k] = acc_s_ref[:]
            r_o = pltpu.make_async_remote_copy(
                src_ref=send_o_ref.at[k], dst_ref=recv_o_ref.at[k],
                send_sem=o_send.at[k], recv_sem=o_recv.at[k],
                device_id=(partners[k],),
                device_id_type=pl.DeviceIdType.MESH)
            r_s = pltpu.make_async_remote_copy(
                src_ref=send_s_ref.at[k], dst_ref=recv_s_ref.at[k],
                send_sem=s_send.at[k], recv_sem=s_recv.at[k],
                device_id=(partners[k],),
                device_id_type=pl.DeviceIdType.MESH)
            r_o.start()
            r_s.start()
            r_o.wait()
            r_s.wait()

            o_r = recv_o_ref[k].astype(jnp.float32)
            m_r = recv_s_ref[k, :, :, 0:1, :]
            l_r = recv_s_ref[k, :, :, 1:2, :]
            m_a = acc_s_ref[:, :, 0:1, :]
            l_a = acc_s_ref[:, :, 1:2, :]
            m_new = jnp.maximum(m_a, m_r)
            a_w = jnp.exp(m_a - m_new)
            b_w = jnp.exp(m_r - m_new)
            acc_o_ref[:] = acc_o_ref[:] * a_w + o_r * b_w
            acc_s_ref[:, :, 0:1, :] = m_new
            acc_s_ref[:, :, 1:2, :] = l_a * a_w + l_r * b_w

        @functools.partial(pl.run_scoped,
                           second_barrier=pltpu.SemaphoreType.REGULAR)
        def _(second_barrier):
            for p in partners:
                pl.semaphore_signal(second_barrier, inc=1, device_id=(p,),
                                    device_id_type=pl.DeviceIdType.MESH)
            pl.semaphore_wait(second_barrier, N_STEPS)

        attn_t = jnp.reshape(
            (acc_o_ref[:] / acc_s_ref[:, :, 1:2, :]).astype(jnp.bfloat16),
            (B, HQ * DH, SQ))
        wo = wo_ref[:].astype(jnp.bfloat16)
        for b in range(B):
            out_ref[b] = lax.dot_general(attn_t[b], wo,
                                         (((0,), (0,)), ((), ())),
                                         preferred_element_type=jnp.float32)

    return pl.pallas_call(
        body,
        out_shape=jax.ShapeDtypeStruct((B, SQ, D), jnp.float32),
        in_specs=[pl.BlockSpec(memory_space=pltpu.VMEM)] * 5,
        out_specs=pl.BlockSpec(memory_space=pltpu.VMEM),
        scratch_shapes=[
            pltpu.VMEM((B, HQ, DH, SQ), jnp.float32),
            pltpu.VMEM((B, HQ, 2, SQ), jnp.float32),
            pltpu.VMEM((N_STEPS, B, HQ, DH, SQ), jnp.bfloat16),
            pltpu.VMEM((N_STEPS, B, HQ, DH, SQ), jnp.bfloat16),
            pltpu.VMEM((N_STEPS, B, HQ, 2, SQ), jnp.float32),
            pltpu.VMEM((N_STEPS, B, HQ, 2, SQ), jnp.float32),
            pltpu.SemaphoreType.DMA((N_STEPS,)),
            pltpu.SemaphoreType.DMA((N_STEPS,)),
            pltpu.SemaphoreType.DMA((N_STEPS,)),
            pltpu.SemaphoreType.DMA((N_STEPS,)),
        ],
        compiler_params=pltpu.CompilerParams(collective_id=0),
    )(x, Wq, Wo, K_ext, V_ext)


# device time: 36113 ns/iter; 2.3769x vs baseline; 1.0345x over previous
import functools

import jax
import jax.numpy as jnp
from jax import lax
from jax.experimental import pallas as pl
from jax.experimental.pallas import tpu as pltpu

N_DEV = 16
N_STEPS = 4
B = 2
SQ = 128
HQ = 8
DH = 64
D = 512


def kernel(x, Wq, Wo, K_ext, V_ext):
    def body(x_ref, wq_ref, wo_ref, k_ref, v_ref, out_ref,
             acc_o_ref, acc_s_ref, send_o_ref, recv_o_ref,
             send_s_ref, recv_s_ref, o_send, o_recv, s_send, s_recv):
        my = lax.axis_index("i")
        partners = [my ^ (1 << k) for k in range(N_STEPS)]

        barrier = pltpu.get_barrier_semaphore()
        for p in partners:
            pl.semaphore_signal(barrier, inc=1, device_id=(p,),
                                device_id_type=pl.DeviceIdType.MESH)

        x2 = jnp.reshape(x_ref[:], (B * SQ, D)).astype(jnp.bfloat16)
        wq = wq_ref[:].astype(jnp.bfloat16)
        q2 = lax.dot_general(x2, wq, (((1,), (0,)), ((), ())),
                             preferred_element_type=jnp.float32)
        q2 = (q2 * 0.125).astype(jnp.bfloat16)

        for b in range(B):
            for h in range(HQ):
                qbh = q2[b * SQ:(b + 1) * SQ, h * DH:(h + 1) * DH]
                kbh = k_ref[b, :, h, :].astype(jnp.bfloat16)
                vbh = v_ref[b, :, h, :].astype(jnp.bfloat16)
                s2 = lax.dot_general(kbh, qbh, (((1,), (1,)), ((), ())),
                                     preferred_element_type=jnp.float32)
                m = jnp.max(s2, axis=0, keepdims=True)
                p2 = jnp.exp(s2 - m)
                l = jnp.sum(p2, axis=0, keepdims=True)
                o_t = lax.dot_general(vbh, p2.astype(jnp.bfloat16),
                                      (((0,), (0,)), ((), ())),
                                      preferred_element_type=jnp.float32)
                acc_o_ref[b, h] = o_t
                acc_s_ref[b, h, 0:1, :] = m
                acc_s_ref[b, h, 1:2, :] = l
                send_o_ref[0, b, h] = o_t.astype(jnp.bfloat16)
                send_s_ref[0, b, h, 0:1, :] = m
                send_s_ref[0, b, h, 1:2, :] = l

        pl.semaphore_wait(barrier, N_STEPS)

        for k in range(N_STEPS):
            r_o = pltpu.make_async_remote_copy(
                src_ref=send_o_ref.at[k], dst_ref=recv_o_ref.at[k],
                send_sem=o_send.at[k], recv_sem=o_recv.at[k],
                device_id=(partners[k],),
                device_id_type=pl.DeviceIdType.MESH)
            r_s = pltpu.make_async_remote_copy(
                src_ref=send_s_ref.at[k], dst_ref=recv_s_ref.at[k],
                send_sem=s_send.at[k], recv_sem=s_recv.at[k],
                device_id=(partners[k],),
                device_id_type=pl.DeviceIdType.MESH)
            r_o.start()
            r_s.start()
            r_o.wait()
            r_s.wait()

            o_r = recv_o_ref[k].astype(jnp.float32)
            m_r = recv_s_ref[k, :, :, 0:1, :]
            l_r = recv_s_ref[k, :, :, 1:2, :]
            m_a = acc_s_ref[:, :, 0:1, :]
            l_a = acc_s_ref[:, :, 1:2, :]
            m_new = jnp.maximum(m_a, m_r)
            a_w = jnp.exp(m_a - m_new)
            b_w = jnp.exp(m_r - m_new)
            o_new = acc_o_ref[:] * a_w + o_r * b_w
            l_new = l_a * a_w + l_r * b_w
            acc_o_ref[:] = o_new
            acc_s_ref[:, :, 0:1, :] = m_new
            acc_s_ref[:, :, 1:2, :] = l_new
            if k + 1 < N_STEPS:
                send_o_ref[k + 1] = o_new.astype(jnp.bfloat16)
                send_s_ref[k + 1, :, :, 0:1, :] = m_new
                send_s_ref[k + 1, :, :, 1:2, :] = l_new

        @functools.partial(pl.run_scoped,
                           second_barrier=pltpu.SemaphoreType.REGULAR)
        def _(second_barrier):
            for p in partners:
                pl.semaphore_signal(second_barrier, inc=1, device_id=(p,),
                                    device_id_type=pl.DeviceIdType.MESH)
            pl.semaphore_wait(second_barrier, N_STEPS)

        attn_t = jnp.reshape(
            (acc_o_ref[:] / acc_s_ref[:, :, 1:2, :]).astype(jnp.bfloat16),
            (B, HQ * DH, SQ))
        wo = wo_ref[:].astype(jnp.bfloat16)
        for b in range(B):
            out_ref[b] = lax.dot_general(attn_t[b], wo,
                                         (((0,), (0,)), ((), ())),
                                         preferred_element_type=jnp.float32)

    return pl.pallas_call(
        body,
        out_shape=jax.ShapeDtypeStruct((B, SQ, D), jnp.float32),
        in_specs=[pl.BlockSpec(memory_space=pltpu.VMEM)] * 5,
        out_specs=pl.BlockSpec(memory_space=pltpu.VMEM),
        scratch_shapes=[
            pltpu.VMEM((B, HQ, DH, SQ), jnp.float32),
            pltpu.VMEM((B, HQ, 2, SQ), jnp.float32),
            pltpu.VMEM((N_STEPS, B, HQ, DH, SQ), jnp.bfloat16),
            pltpu.VMEM((N_STEPS, B, HQ, DH, SQ), jnp.bfloat16),
            pltpu.VMEM((N_STEPS, B, HQ, 2, SQ), jnp.float32),
            pltpu.VMEM((N_STEPS, B, HQ, 2, SQ), jnp.float32),
            pltpu.SemaphoreType.DMA((N_STEPS,)),
            pltpu.SemaphoreType.DMA((N_STEPS,)),
            pltpu.SemaphoreType.DMA((N_STEPS,)),
            pltpu.SemaphoreType.DMA((N_STEPS,)),
        ],
        compiler_params=pltpu.CompilerParams(collective_id=0),
    )(x, Wq, Wo, K_ext, V_ext)


# device time: 28756 ns/iter; 2.9850x vs baseline; 1.2558x over previous
import functools

import jax
import jax.numpy as jnp
from jax import lax
from jax.experimental import pallas as pl
from jax.experimental.pallas import tpu as pltpu

N_DEV = 16
N_STEPS = 4
N_GRP = 2
HG = 4
B = 2
SQ = 128
HQ = 8
DH = 64
D = 512

MASKS = ((1, 2, 4, 8), (4, 8, 1, 2))


def kernel(x, Wq, Wo, K_ext, V_ext):
    def body(x_ref, wq_ref, wo_ref, k_ref, v_ref, out_ref,
             acc_o_ref, acc_s_ref, send_o_ref, recv_o_ref,
             send_s_ref, recv_s_ref, o_send, o_recv, s_send, s_recv):
        my = lax.axis_index("i")
        barrier_partners = [my ^ (1 << k) for k in range(N_STEPS)]

        barrier = pltpu.get_barrier_semaphore()
        for p in barrier_partners:
            pl.semaphore_signal(barrier, inc=1, device_id=(p,),
                                device_id_type=pl.DeviceIdType.MESH)

        x2 = jnp.reshape(x_ref[:], (B * SQ, D)).astype(jnp.bfloat16)
        wq = wq_ref[:].astype(jnp.bfloat16)
        q2 = lax.dot_general(x2, wq, (((1,), (0,)), ((), ())),
                             preferred_element_type=jnp.float32)
        q2 = (q2 * 0.125).astype(jnp.bfloat16)

        for b in range(B):
            for h in range(HQ):
                qbh = q2[b * SQ:(b + 1) * SQ, h * DH:(h + 1) * DH]
                kbh = k_ref[b, :, h, :].astype(jnp.bfloat16)
                vbh = v_ref[b, :, h, :].astype(jnp.bfloat16)
                s2 = lax.dot_general(kbh, qbh, (((1,), (1,)), ((), ())),
                                     preferred_element_type=jnp.float32)
                m = jnp.max(s2, axis=0, keepdims=True)
                p2 = jnp.exp(s2 - m)
                l = jnp.sum(p2, axis=0, keepdims=True)
                o_t = lax.dot_general(vbh, p2.astype(jnp.bfloat16),
                                      (((0,), (0,)), ((), ())),
                                      preferred_element_type=jnp.float32)
                acc_o_ref[b, h] = o_t
                acc_s_ref[b, h, 0:1, :] = m
                acc_s_ref[b, h, 1:2, :] = l
                g, hh = divmod(h, HG)
                send_o_ref[g, 0, b, hh] = o_t.astype(jnp.bfloat16)
                send_s_ref[g, 0, b, hh, 0:1, :] = m
                send_s_ref[g, 0, b, hh, 1:2, :] = l

        pl.semaphore_wait(barrier, N_STEPS)

        rdmas = {}
        for g in range(N_GRP):
            for k in range(N_STEPS):
                partner = my ^ MASKS[g][k]
                rdmas[g, k] = (
                    pltpu.make_async_remote_copy(
                        src_ref=send_o_ref.at[g, k],
                        dst_ref=recv_o_ref.at[g, k],
                        send_sem=o_send.at[g, k], recv_sem=o_recv.at[g, k],
                        device_id=(partner,),
                        device_id_type=pl.DeviceIdType.MESH),
                    pltpu.make_async_remote_copy(
                        src_ref=send_s_ref.at[g, k],
                        dst_ref=recv_s_ref.at[g, k],
                        send_sem=s_send.at[g, k], recv_sem=s_recv.at[g, k],
                        device_id=(partner,),
                        device_id_type=pl.DeviceIdType.MESH),
                )

        def start(g, k):
            rdmas[g, k][0].start()
            rdmas[g, k][1].start()

        def wait(g, k):
            rdmas[g, k][0].wait()
            rdmas[g, k][1].wait()

        def combine(g, k):
            hs = slice(HG * g, HG * (g + 1))
            o_r = recv_o_ref[g, k].astype(jnp.float32)
            m_r = recv_s_ref[g, k, :, :, 0:1, :]
            l_r = recv_s_ref[g, k, :, :, 1:2, :]
            m_a = acc_s_ref[:, hs, 0:1, :]
            l_a = acc_s_ref[:, hs, 1:2, :]
            m_new = jnp.maximum(m_a, m_r)
            a_w = jnp.exp(m_a - m_new)
            b_w = jnp.exp(m_r - m_new)
            o_new = acc_o_ref[:, hs] * a_w + o_r * b_w
            l_new = l_a * a_w + l_r * b_w
            acc_o_ref[:, hs] = o_new
            acc_s_ref[:, hs, 0:1, :] = m_new
            acc_s_ref[:, hs, 1:2, :] = l_new
            if k + 1 < N_STEPS:
                send_o_ref[g, k + 1] = o_new.astype(jnp.bfloat16)
                send_s_ref[g, k + 1, :, :, 0:1, :] = m_new
                send_s_ref[g, k + 1, :, :, 1:2, :] = l_new

        start(0, 0)
        start(1, 0)
        for k in range(N_STEPS):
            wait(0, k)
            combine(0, k)
            if k + 1 < N_STEPS:
                start(0, k + 1)
            wait(1, k)
            combine(1, k)
            if k + 1 < N_STEPS:
                start(1, k + 1)

        @functools.partial(pl.run_scoped,
                           second_barrier=pltpu.SemaphoreType.REGULAR)
        def _(second_barrier):
            for p in barrier_partners:
                pl.semaphore_signal(second_barrier, inc=1, device_id=(p,),
                                    device_id_type=pl.DeviceIdType.MESH)
            pl.semaphore_wait(second_barrier, N_STEPS)

        attn_t = jnp.reshape(
            (acc_o_ref[:] / acc_s_ref[:, :, 1:2, :]).astype(jnp.bfloat16),
            (B, HQ * DH, SQ))
        wo = wo_ref[:].astype(jnp.bfloat16)
        for b in range(B):
            out_ref[b] = lax.dot_general(attn_t[b], wo,
                                         (((0,), (0,)), ((), ())),
                                         preferred_element_type=jnp.float32)

    return pl.pallas_call(
        body,
        out_shape=jax.ShapeDtypeStruct((B, SQ, D), jnp.float32),
        in_specs=[pl.BlockSpec(memory_space=pltpu.VMEM)] * 5,
        out_specs=pl.BlockSpec(memory_space=pltpu.VMEM),
        scratch_shapes=[
            pltpu.VMEM((B, HQ, DH, SQ), jnp.float32),
            pltpu.VMEM((B, HQ, 2, SQ), jnp.float32),
            pltpu.VMEM((N_GRP, N_STEPS, B, HG, DH, SQ), jnp.bfloat16),
            pltpu.VMEM((N_GRP, N_STEPS, B, HG, DH, SQ), jnp.bfloat16),
            pltpu.VMEM((N_GRP, N_STEPS, B, HG, 2, SQ), jnp.float32),
            pltpu.VMEM((N_GRP, N_STEPS, B, HG, 2, SQ), jnp.float32),
            pltpu.SemaphoreType.DMA((N_GRP, N_STEPS)),
            pltpu.SemaphoreType.DMA((N_GRP, N_STEPS)),
            pltpu.SemaphoreType.DMA((N_GRP, N_STEPS)),
            pltpu.SemaphoreType.DMA((N_GRP, N_STEPS)),
        ],
        compiler_params=pltpu.CompilerParams(collective_id=0),
    )(x, Wq, Wo, K_ext, V_ext)


# device time: 27925 ns/iter; 3.0739x vs baseline; 1.0298x over previous
import functools

import jax
import jax.numpy as jnp
from jax import lax
from jax.experimental import pallas as pl
from jax.experimental.pallas import tpu as pltpu

N_DEV = 16
N_STEPS = 4
N_GRP = 2
HG = 4
B = 2
SQ = 128
HQ = 8
DH = 64
D = 512

MASKS = ((1, 2, 4, 8), (4, 8, 1, 2))


def kernel(x, Wq, Wo, K_ext, V_ext):
    def body(x_ref, wq_ref, wo_ref, k_ref, v_ref, out_ref,
             acc_o_ref, acc_s_ref, send_o_ref, recv_o_ref,
             send_s_ref, recv_s_ref, o_send, o_recv, s_send, s_recv):
        my = lax.axis_index("i")
        barrier_partners = [my ^ (1 << k) for k in range(N_STEPS)]

        barrier = pltpu.get_barrier_semaphore()
        for p in barrier_partners:
            pl.semaphore_signal(barrier, inc=1, device_id=(p,),
                                device_id_type=pl.DeviceIdType.MESH)

        rdmas = {}
        for g in range(N_GRP):
            for k in range(N_STEPS):
                partner = my ^ MASKS[g][k]
                rdmas[g, k] = (
                    pltpu.make_async_remote_copy(
                        src_ref=send_o_ref.at[g, k],
                        dst_ref=recv_o_ref.at[g, k],
                        send_sem=o_send.at[g, k], recv_sem=o_recv.at[g, k],
                        device_id=(partner,),
                        device_id_type=pl.DeviceIdType.MESH),
                    pltpu.make_async_remote_copy(
                        src_ref=send_s_ref.at[g, k],
                        dst_ref=recv_s_ref.at[g, k],
                        send_sem=s_send.at[g, k], recv_sem=s_recv.at[g, k],
                        device_id=(partner,),
                        device_id_type=pl.DeviceIdType.MESH),
                )

        def start(g, k):
            rdmas[g, k][0].start()
            rdmas[g, k][1].start()

        def wait(g, k):
            rdmas[g, k][0].wait()
            rdmas[g, k][1].wait()

        x2 = jnp.reshape(x_ref[:], (B * SQ, D)).astype(jnp.bfloat16)
        wq = wq_ref[:].astype(jnp.bfloat16)
        q2 = lax.dot_general(x2, wq, (((1,), (0,)), ((), ())),
                             preferred_element_type=jnp.float32)
        q2 = (q2 * 0.125).astype(jnp.bfloat16)

        for g in range(N_GRP):
            for b in range(B):
                for hh in range(HG):
                    h = HG * g + hh
                    qbh = q2[b * SQ:(b + 1) * SQ, h * DH:(h + 1) * DH]
                    kbh = k_ref[b, :, h, :].astype(jnp.bfloat16)
                    vbh = v_ref[b, :, h, :].astype(jnp.bfloat16)
                    s2 = lax.dot_general(kbh, qbh, (((1,), (1,)), ((), ())),
                                         preferred_element_type=jnp.float32)
                    m = jnp.max(s2, axis=0, keepdims=True)
                    p2 = jnp.exp(s2 - m)
                    l = jnp.sum(p2, axis=0, keepdims=True)
                    o_t = lax.dot_general(vbh, p2.astype(jnp.bfloat16),
                                          (((0,), (0,)), ((), ())),
                                          preferred_element_type=jnp.float32)
                    acc_o_ref[b, h] = o_t
                    acc_s_ref[b, h, 0:1, :] = m
                    acc_s_ref[b, h, 1:2, :] = l
                    send_o_ref[g, 0, b, hh] = o_t.astype(jnp.bfloat16)
                    send_s_ref[g, 0, b, hh, 0:1, :] = m
                    send_s_ref[g, 0, b, hh, 1:2, :] = l
            if g == 0:
                pl.semaphore_wait(barrier, N_STEPS)
            start(g, 0)

        def combine(g, k):
            hs = slice(HG * g, HG * (g + 1))
            o_r = recv_o_ref[g, k].astype(jnp.float32)
            m_r = recv_s_ref[g, k, :, :, 0:1, :]
            l_r = recv_s_ref[g, k, :, :, 1:2, :]
            m_a = acc_s_ref[:, hs, 0:1, :]
            l_a = acc_s_ref[:, hs, 1:2, :]
            m_new = jnp.maximum(m_a, m_r)
            a_w = jnp.exp(m_a - m_new)
            b_w = jnp.exp(m_r - m_new)
            o_new = acc_o_ref[:, hs] * a_w + o_r * b_w
            l_new = l_a * a_w + l_r * b_w
            acc_o_ref[:, hs] = o_new
            acc_s_ref[:, hs, 0:1, :] = m_new
            acc_s_ref[:, hs, 1:2, :] = l_new
            if k + 1 < N_STEPS:
                send_o_ref[g, k + 1] = o_new.astype(jnp.bfloat16)
                send_s_ref[g, k + 1, :, :, 0:1, :] = m_new
                send_s_ref[g, k + 1, :, :, 1:2, :] = l_new

        def proj(g):
            hs = slice(HG * g, HG * (g + 1))
            attn_t = jnp.reshape(
                (acc_o_ref[:, hs] /
                 acc_s_ref[:, hs, 1:2, :]).astype(jnp.bfloat16),
                (B, HG * DH, SQ))
            wo_g = wo_ref[HG * DH * g:HG * DH * (g + 1), :].astype(
                jnp.bfloat16)
            return [lax.dot_general(attn_t[b], wo_g,
                                    (((0,), (0,)), ((), ())),
                                    preferred_element_type=jnp.float32)
                    for b in range(B)]

        for k in range(N_STEPS):
            wait(0, k)
            combine(0, k)
            if k + 1 < N_STEPS:
                start(0, k + 1)
                wait(1, k)
                combine(1, k)
                start(1, k + 1)

        out0 = proj(0)
        wait(1, N_STEPS - 1)
        combine(1, N_STEPS - 1)

        @functools.partial(pl.run_scoped,
                           second_barrier=pltpu.SemaphoreType.REGULAR)
        def _(second_barrier):
            for p in barrier_partners:
                pl.semaphore_signal(second_barrier, inc=1, device_id=(p,),
                                    device_id_type=pl.DeviceIdType.MESH)
            out1 = proj(1)
            for b in range(B):
                out_ref[b] = out0[b] + out1[b]
            pl.semaphore_wait(second_barrier, N_STEPS)

    return pl.pallas_call(
        body,
        out_shape=jax.ShapeDtypeStruct((B, SQ, D), jnp.float32),
        in_specs=[pl.BlockSpec(memory_space=pltpu.VMEM)] * 5,
        out_specs=pl.BlockSpec(memory_space=pltpu.VMEM),
        scratch_shapes=[
            pltpu.VMEM((B, HQ, DH, SQ), jnp.float32),
            pltpu.VMEM((B, HQ, 2, SQ), jnp.float32),
            pltpu.VMEM((N_GRP, N_STEPS, B, HG, DH, SQ), jnp.bfloat16),
            pltpu.VMEM((N_GRP, N_STEPS, B, HG, DH, SQ), jnp.bfloat16),
            pltpu.VMEM((N_GRP, N_STEPS, B, HG, 2, SQ), jnp.float32),
            pltpu.VMEM((N_GRP, N_STEPS, B, HG, 2, SQ), jnp.float32),
            pltpu.SemaphoreType.DMA((N_GRP, N_STEPS)),
            pltpu.SemaphoreType.DMA((N_GRP, N_STEPS)),
            pltpu.SemaphoreType.DMA((N_GRP, N_STEPS)),
            pltpu.SemaphoreType.DMA((N_GRP, N_STEPS)),
        ],
        compiler_params=pltpu.CompilerParams(collective_id=0),
    )(x, Wq, Wo, K_ext, V_ext)


# device time: 27020 ns/iter; 3.1768x vs baseline; 1.0335x over previous
import functools

import jax
import jax.numpy as jnp
from jax import lax
from jax.experimental import pallas as pl
from jax.experimental.pallas import tpu as pltpu

N_DEV = 16
N_STEPS = 4
N_GRP = 2
HG = 4
B = 2
SQ = 128
HQ = 8
DH = 64
D = 512

MASKS = ((1, 3, 4, 8), (4, 8, 1, 3))


def kernel(x, Wq, Wo, K_ext, V_ext):
    def body(x_ref, wq_ref, wo_ref, k_ref, v_ref, out_ref,
             acc_o_ref, acc_s_ref, send_o_ref, recv_o_ref,
             send_s_ref, recv_s_ref, o_send, o_recv, s_send, s_recv):
        my = lax.axis_index("i")
        barrier_partners = [my ^ mask for mask in MASKS[0]]

        barrier = pltpu.get_barrier_semaphore()
        for p in barrier_partners:
            pl.semaphore_signal(barrier, inc=1, device_id=(p,),
                                device_id_type=pl.DeviceIdType.MESH)

        rdmas = {}
        for g in range(N_GRP):
            for k in range(N_STEPS):
                partner = my ^ MASKS[g][k]
                rdmas[g, k] = (
                    pltpu.make_async_remote_copy(
                        src_ref=send_o_ref.at[g, k],
                        dst_ref=recv_o_ref.at[g, k],
                        send_sem=o_send.at[g, k], recv_sem=o_recv.at[g, k],
                        device_id=(partner,),
                        device_id_type=pl.DeviceIdType.MESH),
                    pltpu.make_async_remote_copy(
                        src_ref=send_s_ref.at[g, k],
                        dst_ref=recv_s_ref.at[g, k],
                        send_sem=s_send.at[g, k], recv_sem=s_recv.at[g, k],
                        device_id=(partner,),
                        device_id_type=pl.DeviceIdType.MESH),
                )

        def start(g, k):
            rdmas[g, k][0].start()
            rdmas[g, k][1].start()

        def wait(g, k):
            rdmas[g, k][0].wait()
            rdmas[g, k][1].wait()

        x2 = jnp.reshape(x_ref[:], (B * SQ, D)).astype(jnp.bfloat16)
        wq = wq_ref[:].astype(jnp.bfloat16)
        q2 = lax.dot_general(x2, wq, (((1,), (0,)), ((), ())),
                             preferred_element_type=jnp.float32)
        q2 = (q2 * 0.125).astype(jnp.bfloat16)

        for g in range(N_GRP):
            for b in range(B):
                for hh in range(HG):
                    h = HG * g + hh
                    qbh = q2[b * SQ:(b + 1) * SQ, h * DH:(h + 1) * DH]
                    kbh = k_ref[b, :, h, :].astype(jnp.bfloat16)
                    vbh = v_ref[b, :, h, :].astype(jnp.bfloat16)
                    s2 = lax.dot_general(kbh, qbh, (((1,), (1,)), ((), ())),
                                         preferred_element_type=jnp.float32)
                    m = jnp.max(s2, axis=0, keepdims=True)
                    p2 = jnp.exp(s2 - m)
                    l = jnp.sum(p2, axis=0, keepdims=True)
                    o_t = lax.dot_general(vbh, p2.astype(jnp.bfloat16),
                                          (((0,), (0,)), ((), ())),
                                          preferred_element_type=jnp.float32)
                    acc_o_ref[b, h] = o_t
                    acc_s_ref[b, h, 0:1, :] = m
                    acc_s_ref[b, h, 1:2, :] = l
                    send_o_ref[g, 0, b, hh] = o_t.astype(jnp.bfloat16)
                    send_s_ref[g, 0, b, hh, 0:1, :] = m
                    send_s_ref[g, 0, b, hh, 1:2, :] = l
            if g == 0:
                pl.semaphore_wait(barrier, N_STEPS)
            start(g, 0)

        def combine(g, k):
            hs = slice(HG * g, HG * (g + 1))
            o_r = recv_o_ref[g, k].astype(jnp.float32)
            m_r = recv_s_ref[g, k, :, :, 0:1, :]
            l_r = recv_s_ref[g, k, :, :, 1:2, :]
            m_a = acc_s_ref[:, hs, 0:1, :]
            l_a = acc_s_ref[:, hs, 1:2, :]
            m_new = jnp.maximum(m_a, m_r)
            a_w = jnp.exp(m_a - m_new)
            b_w = jnp.exp(m_r - m_new)
            o_new = acc_o_ref[:, hs] * a_w + o_r * b_w
            l_new = l_a * a_w + l_r * b_w
            acc_o_ref[:, hs] = o_new
            acc_s_ref[:, hs, 0:1, :] = m_new
            acc_s_ref[:, hs, 1:2, :] = l_new
            if k + 1 < N_STEPS:
                send_o_ref[g, k + 1] = o_new.astype(jnp.bfloat16)
                send_s_ref[g, k + 1, :, :, 0:1, :] = m_new
                send_s_ref[g, k + 1, :, :, 1:2, :] = l_new

        def proj(g):
            hs = slice(HG * g, HG * (g + 1))
            attn_t = jnp.reshape(
                (acc_o_ref[:, hs] /
                 acc_s_ref[:, hs, 1:2, :]).astype(jnp.bfloat16),
                (B, HG * DH, SQ))
            wo_g = wo_ref[HG * DH * g:HG * DH * (g + 1), :].astype(
                jnp.bfloat16)
            return [lax.dot_general(attn_t[b], wo_g,
                                    (((0,), (0,)), ((), ())),
                                    preferred_element_type=jnp.float32)
                    for b in range(B)]

        for k in range(N_STEPS):
            wait(0, k)
            combine(0, k)
            if k + 1 < N_STEPS:
                start(0, k + 1)
                wait(1, k)
                combine(1, k)
                start(1, k + 1)

        out0 = proj(0)
        wait(1, N_STEPS - 1)
        combine(1, N_STEPS - 1)

        @functools.partial(pl.run_scoped,
                           second_barrier=pltpu.SemaphoreType.REGULAR)
        def _(second_barrier):
            for p in barrier_partners:
                pl.semaphore_signal(second_barrier, inc=1, device_id=(p,),
                                    device_id_type=pl.DeviceIdType.MESH)
            out1 = proj(1)
            for b in range(B):
                out_ref[b] = out0[b] + out1[b]
            pl.semaphore_wait(second_barrier, N_STEPS)

    return pl.pallas_call(
        body,
        out_shape=jax.ShapeDtypeStruct((B, SQ, D), jnp.float32),
        in_specs=[pl.BlockSpec(memory_space=pltpu.VMEM)] * 5,
        out_specs=pl.BlockSpec(memory_space=pltpu.VMEM),
        scratch_shapes=[
            pltpu.VMEM((B, HQ, DH, SQ), jnp.float32),
            pltpu.VMEM((B, HQ, 2, SQ), jnp.float32),
            pltpu.VMEM((N_GRP, N_STEPS, B, HG, DH, SQ), jnp.bfloat16),
            pltpu.VMEM((N_GRP, N_STEPS, B, HG, DH, SQ), jnp.bfloat16),
            pltpu.VMEM((N_GRP, N_STEPS, B, HG, 2, SQ), jnp.float32),
            pltpu.VMEM((N_GRP, N_STEPS, B, HG, 2, SQ), jnp.float32),
            pltpu.SemaphoreType.DMA((N_GRP, N_STEPS)),
            pltpu.SemaphoreType.DMA((N_GRP, N_STEPS)),
            pltpu.SemaphoreType.DMA((N_GRP, N_STEPS)),
            pltpu.SemaphoreType.DMA((N_GRP, N_STEPS)),
        ],
        compiler_params=pltpu.CompilerParams(collective_id=0),
    )(x, Wq, Wo, K_ext, V_ext)


# device time: 24222 ns/iter; 3.5438x vs baseline; 1.1155x over previous
import functools

import jax
import jax.numpy as jnp
from jax import lax
from jax.experimental import pallas as pl
from jax.experimental.pallas import tpu as pltpu

N_DEV = 16
N_STEPS = 4
N_GRP = 4
HG = 2
B = 2
SQ = 128
HQ = 8
DH = 64
D = 512

BASE_MASKS = (1, 3, 4, 8)
MASKS = tuple(tuple(BASE_MASKS[(k + g) % N_STEPS] for k in range(N_STEPS))
              for g in range(N_GRP))


def kernel(x, Wq, Wo, K_ext, V_ext):
    def body(x_ref, wq_ref, wo_ref, k_ref, v_ref, out_ref,
             acc_o_ref, acc_s_ref, send_o_ref, recv_o_ref,
             send_s_ref, recv_s_ref, o_send, o_recv, s_send, s_recv):
        my = lax.axis_index("i")
        barrier_partners = [my ^ mask for mask in BASE_MASKS]

        barrier = pltpu.get_barrier_semaphore()
        for p in barrier_partners:
            pl.semaphore_signal(barrier, inc=1, device_id=(p,),
                                device_id_type=pl.DeviceIdType.MESH)

        rdmas = {}
        for g in range(N_GRP):
            for k in range(N_STEPS):
                partner = my ^ MASKS[g][k]
                rdmas[g, k] = (
                    pltpu.make_async_remote_copy(
                        src_ref=send_o_ref.at[g, k],
                        dst_ref=recv_o_ref.at[g, k],
                        send_sem=o_send.at[g, k], recv_sem=o_recv.at[g, k],
                        device_id=(partner,),
                        device_id_type=pl.DeviceIdType.MESH),
                    pltpu.make_async_remote_copy(
                        src_ref=send_s_ref.at[g, k],
                        dst_ref=recv_s_ref.at[g, k],
                        send_sem=s_send.at[g, k], recv_sem=s_recv.at[g, k],
                        device_id=(partner,),
                        device_id_type=pl.DeviceIdType.MESH),
                )

        def start(g, k):
            rdmas[g, k][0].start()
            rdmas[g, k][1].start()

        def wait(g, k):
            rdmas[g, k][0].wait()
            rdmas[g, k][1].wait()

        x2 = jnp.reshape(x_ref[:], (B * SQ, D)).astype(jnp.bfloat16)
        wq = wq_ref[:].astype(jnp.bfloat16)
        q2 = lax.dot_general(x2, wq, (((1,), (0,)), ((), ())),
                             preferred_element_type=jnp.float32)
        q2 = (q2 * 0.125).astype(jnp.bfloat16)

        for g in range(N_GRP):
            for b in range(B):
                for hh in range(HG):
                    h = HG * g + hh
                    qbh = q2[b * SQ:(b + 1) * SQ, h * DH:(h + 1) * DH]
                    kbh = k_ref[b, :, h, :].astype(jnp.bfloat16)
                    vbh = v_ref[b, :, h, :].astype(jnp.bfloat16)
                    s2 = lax.dot_general(kbh, qbh, (((1,), (1,)), ((), ())),
                                         preferred_element_type=jnp.float32)
                    m = jnp.max(s2, axis=0, keepdims=True)
                    p2 = jnp.exp(s2 - m)
                    l = jnp.sum(p2, axis=0, keepdims=True)
                    o_t = lax.dot_general(vbh, p2.astype(jnp.bfloat16),
                                          (((0,), (0,)), ((), ())),
                                          preferred_element_type=jnp.float32)
                    acc_o_ref[b, h] = o_t
                    acc_s_ref[b, h, 0:1, :] = m
                    acc_s_ref[b, h, 1:2, :] = l
                    send_o_ref[g, 0, b, hh] = o_t.astype(jnp.bfloat16)
                    send_s_ref[g, 0, b, hh, 0:1, :] = m
                    send_s_ref[g, 0, b, hh, 1:2, :] = l
            if g == 0:
                pl.semaphore_wait(barrier, N_STEPS)
            start(g, 0)

        def combine(g, k):
            hs = slice(HG * g, HG * (g + 1))
            o_r = recv_o_ref[g, k].astype(jnp.float32)
            m_r = recv_s_ref[g, k, :, :, 0:1, :]
            l_r = recv_s_ref[g, k, :, :, 1:2, :]
            m_a = acc_s_ref[:, hs, 0:1, :]
            l_a = acc_s_ref[:, hs, 1:2, :]
            m_new = jnp.maximum(m_a, m_r)
            a_w = jnp.exp(m_a - m_new)
            b_w = jnp.exp(m_r - m_new)
            o_new = acc_o_ref[:, hs] * a_w + o_r * b_w
            l_new = l_a * a_w + l_r * b_w
            acc_o_ref[:, hs] = o_new
            acc_s_ref[:, hs, 0:1, :] = m_new
            acc_s_ref[:, hs, 1:2, :] = l_new
            if k + 1 < N_STEPS:
                send_o_ref[g, k + 1] = o_new.astype(jnp.bfloat16)
                send_s_ref[g, k + 1, :, :, 0:1, :] = m_new
                send_s_ref[g, k + 1, :, :, 1:2, :] = l_new

        def proj(half):
            hs = slice(4 * half, 4 * (half + 1))
            attn_t = jnp.reshape(
                (acc_o_ref[:, hs] /
                 acc_s_ref[:, hs, 1:2, :]).astype(jnp.bfloat16),
                (B, 4 * DH, SQ))
            wo_h = wo_ref[4 * DH * half:4 * DH * (half + 1), :].astype(
                jnp.bfloat16)
            return [lax.dot_general(attn_t[b], wo_h,
                                    (((0,), (0,)), ((), ())),
                                    preferred_element_type=jnp.float32)
                    for b in range(B)]

        for k in range(N_STEPS - 1):
            for g in range(N_GRP):
                wait(g, k)
                combine(g, k)
                start(g, k + 1)

        wait(0, N_STEPS - 1)
        combine(0, N_STEPS - 1)
        wait(1, N_STEPS - 1)
        combine(1, N_STEPS - 1)
        out0 = proj(0)
        wait(2, N_STEPS - 1)
        combine(2, N_STEPS - 1)
        wait(3, N_STEPS - 1)
        combine(3, N_STEPS - 1)

        @functools.partial(pl.run_scoped,
                           second_barrier=pltpu.SemaphoreType.REGULAR)
        def _(second_barrier):
            for p in barrier_partners:
                pl.semaphore_signal(second_barrier, inc=1, device_id=(p,),
                                    device_id_type=pl.DeviceIdType.MESH)
            out1 = proj(1)
            for b in range(B):
                out_ref[b] = out0[b] + out1[b]
            pl.semaphore_wait(second_barrier, N_STEPS)

    return pl.pallas_call(
        body,
        out_shape=jax.ShapeDtypeStruct((B, SQ, D), jnp.float32),
        in_specs=[pl.BlockSpec(memory_space=pltpu.VMEM)] * 5,
        out_specs=pl.BlockSpec(memory_space=pltpu.VMEM),
        scratch_shapes=[
            pltpu.VMEM((B, HQ, DH, SQ), jnp.float32),
            pltpu.VMEM((B, HQ, 2, SQ), jnp.float32),
            pltpu.VMEM((N_GRP, N_STEPS, B, HG, DH, SQ), jnp.bfloat16),
            pltpu.VMEM((N_GRP, N_STEPS, B, HG, DH, SQ), jnp.bfloat16),
            pltpu.VMEM((N_GRP, N_STEPS, B, HG, 2, SQ), jnp.float32),
            pltpu.VMEM((N_GRP, N_STEPS, B, HG, 2, SQ), jnp.float32),
            pltpu.SemaphoreType.DMA((N_GRP, N_STEPS)),
            pltpu.SemaphoreType.DMA((N_GRP, N_STEPS)),
            pltpu.SemaphoreType.DMA((N_GRP, N_STEPS)),
            pltpu.SemaphoreType.DMA((N_GRP, N_STEPS)),
        ],
        compiler_params=pltpu.CompilerParams(collective_id=0),
    )(x, Wq, Wo, K_ext, V_ext)
